# baseline (speedup 1.0000x reference)
import functools

import jax
import jax.numpy as jnp
from jax import lax
from jax.experimental import pallas as pl
from jax.experimental.pallas import tpu as pltpu
from jax.experimental.pallas import tpu_sc as plsc

_NC = 2
_NS = 16
_E_CHUNK = 64
_EPW = 10240


def _proj_body(eat_ref, we_ref, be_ref, o_ref, *, e_real, blk):
    i = pl.program_id(0)
    acc = lax.dot_general(
        eat_ref[...], we_ref[...], (((0,), (0,)), ((), ())),
        preferred_element_type=jnp.float32,
        precision=lax.Precision.DEFAULT,
    ) + be_ref[...]
    row = lax.broadcasted_iota(jnp.int32, (blk, 1), 0) + i * blk
    acc = jnp.where(row >= e_real, jnp.float32(-1e30), acc)
    o_ref[...] = acc.astype(jnp.bfloat16)


def _project(edge_attr_t, We, be, e_pad):
    K, E = edge_attr_t.shape
    D = We.shape[1]
    BLK = 8192
    eat_p = jnp.pad(edge_attr_t, ((0, 0), (0, e_pad - E)))
    return pl.pallas_call(
        functools.partial(_proj_body, e_real=E, blk=BLK),
        grid=(e_pad // BLK,),
        in_specs=[
            pl.BlockSpec((K, BLK), lambda i: (0, i)),
            pl.BlockSpec((K, D), lambda i: (0, 0)),
            pl.BlockSpec((D,), lambda i: (0,)),
        ],
        out_specs=pl.BlockSpec((BLK, D), lambda i: (i, 0)),
        out_shape=jax.ShapeDtypeStruct((e_pad, D), jnp.bfloat16),
    )(eat_p, We, be)


_NBUF = 2
_NSRC = 4
_NDST = 8


def _sc_aggregate(x, src, dst, e_i):
    N, D = x.shape
    n_w = _NC * _NS
    e_per_w = src.shape[0] // n_w
    K = _E_CHUNK
    n_chunks = e_per_w // K
    KH = K // 2
    ZROWS = 128
    rows_per_tile = -(-N // (_NS * ZROWS)) * ZROWS
    NP = _NS * rows_per_tile
    BROWS = _NBUF * K

    mesh = plsc.VectorSubcoreMesh(core_axis_name="c", subcore_axis_name="s")

    @functools.partial(
        pl.kernel,
        out_type=jax.ShapeDtypeStruct((_NC, NP, D), jnp.float32),
        mesh=mesh,
        scratch_types=dict(
            aggr_sh=pltpu.VMEM_SHARED((NP, D), jnp.float32),
            src_v=pltpu.VMEM((_NSRC, K), jnp.int32),
            dst_v=pltpu.VMEM((_NDST, K), jnp.int32),
            e_iv=pltpu.VMEM((_NBUF * KH, D), jnp.int32),
            xg_v=pltpu.VMEM((BROWS, D), jnp.float32),
            m_v=pltpu.VMEM((BROWS, D), jnp.float32),
            sem_idx=pltpu.SemaphoreType.DMA((_NSRC,)),
            sem_in=pltpu.SemaphoreType.DMA((_NBUF,)),
            sem_sc=pltpu.SemaphoreType.DMA((_NBUF,)),
        ),
    )
    def k(x_hbm, src_hbm, dst_hbm, ei_hbm, out_hbm,
          aggr_sh, src_v, dst_v, e_iv, xg_v, m_v, sem_idx, sem_in, sem_sc):
        cid = lax.axis_index("c")
        sid = lax.axis_index("s")
        wid = cid * _NS + sid

        zero = jnp.zeros((16,), jnp.float32)

        def zrow(r, carry):
            for j in range(D // 16):
                m_v[r, pl.ds(j * 16, 16)] = zero
            return carry

        lax.fori_loop(0, BROWS, zrow, 0)
        r0 = sid * rows_per_tile
        done = 0
        while done < rows_per_tile:
            n = min(BROWS, rows_per_tile - done)
            pltpu.sync_copy(m_v.at[pl.ds(0, n), :],
                            aggr_sh.at[pl.ds(r0 + done, n), :])
            done += n
        plsc.subcore_barrier()

        def e_slab(b):
            return e_iv.at[pl.ds(b * KH, KH), :]

        def xg_slab(b):
            return xg_v.at[pl.ds(b * K, K), :]

        def m_slab(b):
            return m_v.at[pl.ds(b * K, K), :]

        def issue_idx(ci):
            base = pl.multiple_of(wid * e_per_w + ci * K, 8)
            q = lax.rem(ci, _NSRC)
            pltpu.async_copy(src_hbm.at[pl.ds(base, K)],
                             src_v.at[q], sem_idx.at[q])
            pltpu.async_copy(dst_hbm.at[pl.ds(base, K)],
                             dst_v.at[lax.rem(ci, _NDST)], sem_idx.at[q])

        def wait_idx(ci):
            base = pl.multiple_of(wid * e_per_w + ci * K, 8)
            q = lax.rem(ci, _NSRC)
            pltpu.make_async_copy(src_hbm.at[pl.ds(base, K)],
                                  src_v.at[q], sem_idx.at[q]).wait()
            pltpu.make_async_copy(dst_hbm.at[pl.ds(base, K)],
                                  dst_v.at[lax.rem(ci, _NDST)],
                                  sem_idx.at[q]).wait()

        def issue_ge(ci, b):
            base_i = pl.multiple_of((wid * e_per_w + ci * K) // 2, 8)
            q = lax.rem(ci, _NSRC)
            pltpu.async_copy(ei_hbm.at[pl.ds(base_i, KH), :], e_slab(b),
                             sem_in.at[b])
            pltpu.async_copy(x_hbm.at[src_v.at[q]], xg_slab(b), sem_in.at[b])

        def wait_ge(ci, b):
            base_i = pl.multiple_of((wid * e_per_w + ci * K) // 2, 8)
            q = lax.rem(ci, _NSRC)
            pltpu.make_async_copy(ei_hbm.at[pl.ds(base_i, KH), :], e_slab(b),
                                  sem_in.at[b]).wait()
            pltpu.make_async_copy(x_hbm.at[src_v.at[q]], xg_slab(b),
                                  sem_in.at[b]).wait()

        def issue_scatter(ci, b):
            pltpu.async_copy(m_slab(b), aggr_sh.at[dst_v.at[lax.rem(ci, _NDST)]],
                             sem_sc.at[b], add=True)

        def wait_scatter(b):
            pltpu.make_async_copy(m_slab(b), aggr_sh.at[dst_v.at[0]],
                                  sem_sc.at[b]).wait()

        himask = jnp.full((16,), -65536, jnp.int32)

        def compute(b):
            def pair(p, carry):
                rp = b * KH + p
                for half in range(2):
                    xr = b * K + 2 * p + half
                    for c in range(D // 32):
                        w = e_iv[rp, pl.ds(half * (D // 2) + c * 16, 16)]
                        elo = lax.bitcast_convert_type(jnp.left_shift(w, 16), jnp.float32)
                        ehi = lax.bitcast_convert_type(w & himask, jnp.float32)
                        xlo = xg_v[xr, pl.ds(c * 32, 16)]
                        xhi = xg_v[xr, pl.ds(c * 32 + 16, 16)]
                        m_v[xr, pl.ds(c * 32, 16)] = jnp.maximum(
                            xlo + elo, 0.0)
                        m_v[xr, pl.ds(c * 32 + 16, 16)] = jnp.maximum(
                            xhi + ehi, 0.0)
                return carry

            lax.fori_loop(0, KH, pair, 0)

        issue_idx(0)
        issue_idx(1)
        issue_idx(2)
        wait_idx(0)
        issue_ge(0, 0)

        def step(ci, b):
            wait_ge(ci, b)

            @pl.when(ci >= _NBUF)
            def _():
                wait_scatter(b)

            compute(b)
            issue_scatter(ci, b)

            @pl.when(ci + 1 < n_chunks)
            def _():
                wait_idx(ci + 1)
                issue_ge(ci + 1, 1 - b)

            @pl.when(ci + 3 < n_chunks)
            def _():
                issue_idx(ci + 3)

        def group(gi, carry):
            for b in range(_NBUF):
                step(gi * _NBUF + b, b)
            return carry

        n_groups = n_chunks // _NBUF
        lax.fori_loop(0, n_groups, group, 0)

        for b in range(_NBUF):
            wait_scatter(b)
        plsc.subcore_barrier()

        done = 0
        while done < rows_per_tile:
            n = min(BROWS, rows_per_tile - done)
            pltpu.sync_copy(aggr_sh.at[pl.ds(r0 + done, n), :],
                            out_hbm.at[cid, pl.ds(r0 + done, n), :])
            done += n

    return k(x, src, dst, e_i)


def _mlp_body(x_ref, a_ref, w1_ref, b1_ref, w2_ref, b2_ref, o_ref):
    h = x_ref[...] + a_ref[0] + a_ref[1]
    t = lax.dot_general(
        h, w1_ref[...], (((1,), (0,)), ((), ())),
        preferred_element_type=jnp.float32,
        precision=lax.Precision.DEFAULT,
    ) + b1_ref[...]
    g = t * 0.5 * (1.0 + lax.erf(t * 0.7071067811865476))
    o_ref[...] = lax.dot_general(
        g, w2_ref[...], (((1,), (0,)), ((), ())),
        preferred_element_type=jnp.float32,
        precision=lax.Precision.DEFAULT,
    ) + b2_ref[...]


def _mlp(x, aggr, W1, b1, W2, b2):
    N, D = x.shape
    H = W1.shape[1]
    BLK = 2000
    return pl.pallas_call(
        _mlp_body,
        grid=(N // BLK,),
        in_specs=[
            pl.BlockSpec((BLK, D), lambda i: (i, 0)),
            pl.BlockSpec((2, BLK, D), lambda i: (0, i, 0)),
            pl.BlockSpec((D, H), lambda i: (0, 0)),
            pl.BlockSpec((H,), lambda i: (0,)),
            pl.BlockSpec((H, H), lambda i: (0, 0)),
            pl.BlockSpec((H,), lambda i: (0,)),
        ],
        out_specs=pl.BlockSpec((BLK, H), lambda i: (i, 0)),
        out_shape=jax.ShapeDtypeStruct((N, H), jnp.float32),
    )(x, aggr, W1, b1, W2, b2)


def _perm(D):
    p = []
    for c in range(D // 32):
        p.extend(c * 32 + 2 * k for k in range(16))
        p.extend(c * 32 + 2 * k + 1 for k in range(16))
    return p


def kernel(x, edge_index, edge_attr, We, be, W1, b1, W2, b2):
    E = edge_index.shape[1]
    N, D = x.shape
    n_w = _NC * _NS
    e_pad = n_w * _EPW
    src = jnp.pad(edge_index[0].astype(jnp.int32), (0, e_pad - E))
    dst = jnp.pad(edge_index[1].astype(jnp.int32), (0, e_pad - E))
    e_bf = _project(edge_attr.T, We, be, e_pad)
    e_i = lax.bitcast_convert_type(
        e_bf.reshape(e_pad, D // 2, 2), jnp.int32).reshape(e_pad // 2, D)
    p = jnp.asarray(_perm(D), dtype=jnp.int32)
    xp = jnp.take(x, p, axis=1)
    W1p = jnp.take(W1, p, axis=0)
    aggr = _sc_aggregate(xp, src, dst, e_i)
    return _mlp(xp, aggr, W1p, b1, W2, b2)


# speedup vs baseline: 6.1796x; 6.1796x over previous
import functools

import jax
import jax.numpy as jnp
from jax import lax
from jax.experimental import pallas as pl
from jax.experimental.pallas import tpu as pltpu
from jax.experimental.pallas import tpu_sc as plsc

_NC = 2
_NS = 16
_E_CHUNK = 40


def _proj_body(eat_ref, we_ref, be_ref, o_ref):
    acc = lax.dot_general(
        eat_ref[...], we_ref[...], (((0,), (0,)), ((), ())),
        preferred_element_type=jnp.float32,
        precision=lax.Precision.DEFAULT,
    )
    o_ref[...] = acc + be_ref[...]


def _project(edge_attr_t, We, be):
    K, E = edge_attr_t.shape
    D = We.shape[1]
    BLK = 12800
    return pl.pallas_call(
        _proj_body,
        grid=(E // BLK,),
        in_specs=[
            pl.BlockSpec((K, BLK), lambda i: (0, i)),
            pl.BlockSpec((K, D), lambda i: (0, 0)),
            pl.BlockSpec((D,), lambda i: (0,)),
        ],
        out_specs=pl.BlockSpec((BLK, D), lambda i: (i, 0)),
        out_shape=jax.ShapeDtypeStruct((E, D), jnp.float32),
    )(edge_attr_t, We, be)


_NBUF = 2
_NDST = 4


def _sc_aggregate(x, src3, dst, e):
    N, D = x.shape
    e_per_w = src3.shape[1]
    n_chunks = e_per_w // _E_CHUNK
    K = _E_CHUNK
    ZROWS = 128
    rows_per_tile = -(-N // (_NS * ZROWS)) * ZROWS
    NP = _NS * rows_per_tile
    BROWS = _NBUF * K

    mesh = plsc.VectorSubcoreMesh(core_axis_name="c", subcore_axis_name="s")

    @functools.partial(
        pl.kernel,
        out_type=jax.ShapeDtypeStruct((_NC, NP, D), jnp.float32),
        mesh=mesh,
        scratch_types=dict(
            aggr_sh=pltpu.VMEM_SHARED((NP, D), jnp.float32),
            src_all=pltpu.VMEM((e_per_w,), jnp.int32),
            dst_v=pltpu.VMEM((_NDST, K), jnp.int32),
            e_v=pltpu.VMEM((BROWS, D), jnp.float32),
            xg_v=pltpu.VMEM((BROWS, D), jnp.float32),
            m_v=pltpu.VMEM((BROWS, D), jnp.float32),
            sem_in=pltpu.SemaphoreType.DMA((_NBUF,)),
            sem_sc=pltpu.SemaphoreType.DMA((_NBUF,)),
        ),
    )
    def k(x_hbm, src3_hbm, dst_hbm, e_hbm, out_hbm,
          aggr_sh, src_all, dst_v, e_v, xg_v, m_v, sem_in, sem_sc):
        cid = lax.axis_index("c")
        sid = lax.axis_index("s")
        wid = cid * _NS + sid

        pltpu.sync_copy(src3_hbm.at[wid], src_all)

        zero = jnp.zeros((16,), jnp.float32)

        def zrow(r, carry):
            for j in range(D // 16):
                m_v[r, pl.ds(j * 16, 16)] = zero
            return carry

        lax.fori_loop(0, BROWS, zrow, 0)
        r0 = sid * rows_per_tile
        done = 0
        while done < rows_per_tile:
            n = min(BROWS, rows_per_tile - done)
            pltpu.sync_copy(m_v.at[pl.ds(0, n), :],
                            aggr_sh.at[pl.ds(r0 + done, n), :])
            done += n
        plsc.subcore_barrier()

        def e_slab(b):
            return e_v.at[pl.ds(b * K, K), :]

        def xg_slab(b):
            return xg_v.at[pl.ds(b * K, K), :]

        def m_slab(b):
            return m_v.at[pl.ds(b * K, K), :]

        def issue_in(ci, b):
            base = pl.multiple_of(wid * e_per_w + ci * K, 8)
            off = pl.multiple_of(ci * K, 8)
            pltpu.async_copy(e_hbm.at[pl.ds(base, K), :], e_slab(b),
                             sem_in.at[b])
            pltpu.async_copy(x_hbm.at[src_all.at[pl.ds(off, K)]], xg_slab(b),
                             sem_in.at[b])
            pltpu.async_copy(dst_hbm.at[pl.ds(base, K)],
                             dst_v.at[lax.rem(ci, _NDST)], sem_in.at[b])

        def wait_in(ci, b):
            base = pl.multiple_of(wid * e_per_w + ci * K, 8)
            off = pl.multiple_of(ci * K, 8)
            pltpu.make_async_copy(e_hbm.at[pl.ds(base, K), :], e_slab(b),
                                  sem_in.at[b]).wait()
            pltpu.make_async_copy(x_hbm.at[src_all.at[pl.ds(off, K)]],
                                  xg_slab(b), sem_in.at[b]).wait()
            pltpu.make_async_copy(dst_hbm.at[pl.ds(base, K)],
                                  dst_v.at[lax.rem(ci, _NDST)],
                                  sem_in.at[b]).wait()

        def issue_scatter(ci, b):
            pltpu.async_copy(m_slab(b), aggr_sh.at[dst_v.at[lax.rem(ci, _NDST)]],
                             sem_sc.at[b], add=True)

        def wait_scatter(b):
            pltpu.make_async_copy(m_slab(b), aggr_sh.at[dst_v.at[0]],
                                  sem_sc.at[b]).wait()

        def compute(b):
            def row(r, carry):
                rr = b * K + r
                for j in range(D // 16):
                    a = xg_v[rr, pl.ds(j * 16, 16)]
                    bb = e_v[rr, pl.ds(j * 16, 16)]
                    m_v[rr, pl.ds(j * 16, 16)] = jnp.maximum(a + bb, 0.0)
                return carry

            lax.fori_loop(0, K, row, 0)

        for b in range(_NBUF):
            issue_in(b, b)

        n_groups = n_chunks // _NBUF

        def group(gi, carry):
            for b in range(_NBUF):
                ci = gi * _NBUF + b
                wait_in(ci, b)

                @pl.when(gi > 0)
                def _():
                    wait_scatter(b)

                compute(b)
                issue_scatter(ci, b)
                nci = ci + _NBUF

                @pl.when(nci < n_chunks)
                def _():
                    issue_in(nci, b)

            return carry

        lax.fori_loop(0, n_groups, group, 0)

        for b in range(_NBUF):
            wait_scatter(b)
        plsc.subcore_barrier()

        done = 0
        while done < rows_per_tile:
            n = min(BROWS, rows_per_tile - done)
            pltpu.sync_copy(aggr_sh.at[pl.ds(r0 + done, n), :],
                            out_hbm.at[cid, pl.ds(r0 + done, n), :])
            done += n

    return k(x, src3, dst, e)


def _mlp_body(x_ref, a0_ref, a1_ref, w1_ref, b1_ref, w2_ref, b2_ref, o_ref):
    h = x_ref[...] + a0_ref[...] + a1_ref[...]
    t = lax.dot_general(
        h, w1_ref[...], (((1,), (0,)), ((), ())),
        preferred_element_type=jnp.float32,
        precision=lax.Precision.DEFAULT,
    ) + b1_ref[...]
    g = t * 0.5 * (1.0 + lax.erf(t * 0.7071067811865476))
    o_ref[...] = lax.dot_general(
        g, w2_ref[...], (((1,), (0,)), ((), ())),
        preferred_element_type=jnp.float32,
        precision=lax.Precision.DEFAULT,
    ) + b2_ref[...]


def _mlp(x, a0, a1, W1, b1, W2, b2):
    N, D = x.shape
    H = W1.shape[1]
    BLK = 2000
    return pl.pallas_call(
        _mlp_body,
        grid=(N // BLK,),
        in_specs=[
            pl.BlockSpec((BLK, D), lambda i: (i, 0)),
            pl.BlockSpec((BLK, D), lambda i: (i, 0)),
            pl.BlockSpec((BLK, D), lambda i: (i, 0)),
            pl.BlockSpec((D, H), lambda i: (0, 0)),
            pl.BlockSpec((H,), lambda i: (0,)),
            pl.BlockSpec((H, H), lambda i: (0, 0)),
            pl.BlockSpec((H,), lambda i: (0,)),
        ],
        out_specs=pl.BlockSpec((BLK, H), lambda i: (i, 0)),
        out_shape=jax.ShapeDtypeStruct((N, H), jnp.float32),
    )(x, a0, a1, W1, b1, W2, b2)


def kernel(x, edge_index, edge_attr, We, be, W1, b1, W2, b2):
    E = edge_index.shape[1]
    n_w = _NC * _NS
    e_per_w = E // n_w
    src3 = edge_index[0].astype(jnp.int32).reshape(n_w, e_per_w)
    dst = edge_index[1].astype(jnp.int32)
    e = _project(edge_attr.T, We, be)
    aggr = _sc_aggregate(x, src3, dst, e)
    n = x.shape[0]
    return _mlp(x, aggr[0, :n], aggr[1, :n], W1, b1, W2, b2)
